# baseline (speedup 1.0000x reference)
import jax
import jax.numpy as jnp
from jax.experimental import pallas as pl
from jax.experimental.pallas import tpu as pltpu

CH = 256


def _body(x_hbm, p_hbm, o_hbm, pvmem, xbuf, obuf, xsem, osem, psem):
    N, D = x_hbm.shape
    S = p_hbm.shape[0]
    nch = N // CH
    pch = S // CH

    def p_load(j):
        return pltpu.make_async_copy(
            p_hbm.at[pl.ds(j * CH, CH)], pvmem.at[pl.ds(j * CH, CH)], psem.at[j]
        )

    def x_load(i, slot):
        return pltpu.make_async_copy(
            x_hbm.at[pl.ds(i * CH, CH)], xbuf.at[slot], xsem.at[slot]
        )

    def o_store(i, slot):
        return pltpu.make_async_copy(
            obuf.at[slot], o_hbm.at[pl.ds(i * CH, CH)], osem.at[slot]
        )

    for j in range(pch):
        p_load(j).start()
    x_load(0, 0).start()

    for i in range(nch):
        slot = i % 2
        if i + 1 < nch:
            x_load(i + 1, (i + 1) % 2).start()
        x_load(i, slot).wait()
        if i < pch:
            p_load(i).wait()
        if i >= 2:
            o_store(i - 2, slot).wait()
        prow = (i * CH) % S
        obuf[slot] = xbuf[slot] + pvmem[pl.ds(prow, CH), :]
        o_store(i, slot).start()

    o_store(nch - 2, nch % 2).wait()
    o_store(nch - 1, (nch - 1) % 2).wait()


def kernel(inputs, pos_table):
    B, S, D = inputs.shape
    x = inputs.reshape(B * S, D)
    out = pl.pallas_call(
        _body,
        in_specs=[
            pl.BlockSpec(memory_space=pl.ANY),
            pl.BlockSpec(memory_space=pl.ANY),
        ],
        out_specs=pl.BlockSpec(memory_space=pl.ANY),
        out_shape=jax.ShapeDtypeStruct((B * S, D), inputs.dtype),
        scratch_shapes=[
            pltpu.VMEM((S, D), jnp.float32),
            pltpu.VMEM((2, CH, D), jnp.float32),
            pltpu.VMEM((2, CH, D), jnp.float32),
            pltpu.SemaphoreType.DMA((2,)),
            pltpu.SemaphoreType.DMA((2,)),
            pltpu.SemaphoreType.DMA((S // CH,)),
        ],
    )(x, pos_table)
    return out.reshape(B, S, D)


# speedup vs baseline: 1.3702x; 1.3702x over previous
import jax
import jax.numpy as jnp
from jax.experimental import pallas as pl
from jax.experimental.pallas import tpu as pltpu

CH = 256
NBUF = 4


def _body(x_hbm, p_hbm, o_hbm, pvmem, xbuf, obuf, xsem, osem, psem):
    N, D = x_hbm.shape
    S = p_hbm.shape[0]
    nch = N // CH
    pch = S // CH

    def p_load(j):
        return pltpu.make_async_copy(
            p_hbm.at[pl.ds(j * CH, CH)], pvmem.at[pl.ds(j * CH, CH)], psem.at[j]
        )

    def x_load(i):
        return pltpu.make_async_copy(
            x_hbm.at[pl.ds(i * CH, CH)], xbuf.at[i % NBUF], xsem.at[i % NBUF]
        )

    def o_store(i):
        return pltpu.make_async_copy(
            obuf.at[i % NBUF], o_hbm.at[pl.ds(i * CH, CH)], osem.at[i % NBUF]
        )

    for j in range(pch):
        p_load(j).start()
    for i in range(NBUF - 1):
        x_load(i).start()

    for i in range(nch):
        if i + NBUF - 1 < nch:
            x_load(i + NBUF - 1).start()
        x_load(i).wait()
        if i < pch:
            p_load(i).wait()
        if i >= NBUF:
            o_store(i - NBUF).wait()
        prow = (i * CH) % S
        obuf[i % NBUF] = xbuf[i % NBUF] + pvmem[pl.ds(prow, CH), :]
        o_store(i).start()

    for i in range(nch - NBUF, nch):
        o_store(i).wait()


def kernel(inputs, pos_table):
    B, S, D = inputs.shape
    x = inputs.reshape(B * S, D)
    out = pl.pallas_call(
        _body,
        in_specs=[
            pl.BlockSpec(memory_space=pl.ANY),
            pl.BlockSpec(memory_space=pl.ANY),
        ],
        out_specs=pl.BlockSpec(memory_space=pl.ANY),
        out_shape=jax.ShapeDtypeStruct((B * S, D), inputs.dtype),
        scratch_shapes=[
            pltpu.VMEM((S, D), jnp.float32),
            pltpu.VMEM((NBUF, CH, D), jnp.float32),
            pltpu.VMEM((NBUF, CH, D), jnp.float32),
            pltpu.SemaphoreType.DMA((NBUF,)),
            pltpu.SemaphoreType.DMA((NBUF,)),
            pltpu.SemaphoreType.DMA((S // CH,)),
        ],
    )(x, pos_table)
    return out.reshape(B, S, D)


# speedup vs baseline: 1.4151x; 1.0328x over previous
import jax
import jax.numpy as jnp
from jax.experimental import pallas as pl
from jax.experimental.pallas import tpu as pltpu

CH = 256
NBUF = 8


def _body(x_hbm, p_hbm, o_hbm, pvmem, xbuf, obuf, xsem, osem, psem):
    N, D = x_hbm.shape
    S = p_hbm.shape[0]
    nch = N // CH
    pch = S // CH

    def p_load(j):
        return pltpu.make_async_copy(
            p_hbm.at[pl.ds(j * CH, CH)], pvmem.at[pl.ds(j * CH, CH)], psem.at[j]
        )

    def x_load(i):
        return pltpu.make_async_copy(
            x_hbm.at[pl.ds(i * CH, CH)], xbuf.at[i % NBUF], xsem.at[i % NBUF]
        )

    def o_store(i):
        return pltpu.make_async_copy(
            obuf.at[i % NBUF], o_hbm.at[pl.ds(i * CH, CH)], osem.at[i % NBUF]
        )

    for j in range(pch):
        p_load(j).start()
    for i in range(NBUF - 1):
        x_load(i).start()

    for i in range(nch):
        if i + NBUF - 1 < nch:
            x_load(i + NBUF - 1).start()
        x_load(i).wait()
        if i < pch:
            p_load(i).wait()
        if i >= NBUF:
            o_store(i - NBUF).wait()
        prow = (i * CH) % S
        obuf[i % NBUF] = xbuf[i % NBUF] + pvmem[pl.ds(prow, CH), :]
        o_store(i).start()

    for i in range(nch - NBUF, nch):
        o_store(i).wait()


def kernel(inputs, pos_table):
    B, S, D = inputs.shape
    x = inputs.reshape(B * S, D)
    out = pl.pallas_call(
        _body,
        in_specs=[
            pl.BlockSpec(memory_space=pl.ANY),
            pl.BlockSpec(memory_space=pl.ANY),
        ],
        out_specs=pl.BlockSpec(memory_space=pl.ANY),
        out_shape=jax.ShapeDtypeStruct((B * S, D), inputs.dtype),
        scratch_shapes=[
            pltpu.VMEM((S, D), jnp.float32),
            pltpu.VMEM((NBUF, CH, D), jnp.float32),
            pltpu.VMEM((NBUF, CH, D), jnp.float32),
            pltpu.SemaphoreType.DMA((NBUF,)),
            pltpu.SemaphoreType.DMA((NBUF,)),
            pltpu.SemaphoreType.DMA((S // CH,)),
        ],
    )(x, pos_table)
    return out.reshape(B, S, D)


# speedup vs baseline: 1.4695x; 1.0385x over previous
import jax
import jax.numpy as jnp
from jax.experimental import pallas as pl
from jax.experimental.pallas import tpu as pltpu

CH = 512
NBUF = 6


def _body(x_hbm, p_hbm, o_hbm, pvmem, xbuf, obuf, xsem, osem, psem):
    N, D = x_hbm.shape
    S = p_hbm.shape[0]
    nch = N // CH
    pch = S // CH

    def p_load(j):
        return pltpu.make_async_copy(
            p_hbm.at[pl.ds(j * CH, CH)], pvmem.at[pl.ds(j * CH, CH)], psem.at[j]
        )

    def x_load(i):
        return pltpu.make_async_copy(
            x_hbm.at[pl.ds(i * CH, CH)], xbuf.at[i % NBUF], xsem.at[i % NBUF]
        )

    def o_store(i):
        return pltpu.make_async_copy(
            obuf.at[i % NBUF], o_hbm.at[pl.ds(i * CH, CH)], osem.at[i % NBUF]
        )

    for j in range(pch):
        p_load(j).start()
    for i in range(NBUF - 1):
        x_load(i).start()

    for i in range(nch):
        if i + NBUF - 1 < nch:
            x_load(i + NBUF - 1).start()
        x_load(i).wait()
        if i < pch:
            p_load(i).wait()
        if i >= NBUF:
            o_store(i - NBUF).wait()
        prow = (i * CH) % S
        obuf[i % NBUF] = xbuf[i % NBUF] + pvmem[pl.ds(prow, CH), :]
        o_store(i).start()

    for i in range(nch - NBUF, nch):
        o_store(i).wait()


def kernel(inputs, pos_table):
    B, S, D = inputs.shape
    x = inputs.reshape(B * S, D)
    out = pl.pallas_call(
        _body,
        in_specs=[
            pl.BlockSpec(memory_space=pl.ANY),
            pl.BlockSpec(memory_space=pl.ANY),
        ],
        out_specs=pl.BlockSpec(memory_space=pl.ANY),
        out_shape=jax.ShapeDtypeStruct((B * S, D), inputs.dtype),
        scratch_shapes=[
            pltpu.VMEM((S, D), jnp.float32),
            pltpu.VMEM((NBUF, CH, D), jnp.float32),
            pltpu.VMEM((NBUF, CH, D), jnp.float32),
            pltpu.SemaphoreType.DMA((NBUF,)),
            pltpu.SemaphoreType.DMA((NBUF,)),
            pltpu.SemaphoreType.DMA((S // CH,)),
        ],
    )(x, pos_table)
    return out.reshape(B, S, D)


# speedup vs baseline: 1.4913x; 1.0149x over previous
import jax
import jax.numpy as jnp
from jax.experimental import pallas as pl
from jax.experimental.pallas import tpu as pltpu

CH = 512
NBUF = 8


def _body(x_hbm, p_hbm, o_hbm, pvmem, xbuf, obuf, xsem, osem, psem):
    N, D = x_hbm.shape
    S = p_hbm.shape[0]
    nch = N // CH
    pch = S // CH

    def p_load(j):
        return pltpu.make_async_copy(
            p_hbm.at[pl.ds(j * CH, CH)], pvmem.at[pl.ds(j * CH, CH)], psem.at[j]
        )

    def x_load(i):
        return pltpu.make_async_copy(
            x_hbm.at[pl.ds(i * CH, CH)], xbuf.at[i % NBUF], xsem.at[i % NBUF]
        )

    def o_store(i):
        return pltpu.make_async_copy(
            obuf.at[i % NBUF], o_hbm.at[pl.ds(i * CH, CH)], osem.at[i % NBUF]
        )

    for j in range(pch):
        p_load(j).start()
    for i in range(NBUF - 1):
        x_load(i).start()

    for i in range(nch):
        if i + NBUF - 1 < nch:
            x_load(i + NBUF - 1).start()
        x_load(i).wait()
        if i < pch:
            p_load(i).wait()
        if i >= NBUF:
            o_store(i - NBUF).wait()
        prow = (i * CH) % S
        obuf[i % NBUF] = xbuf[i % NBUF] + pvmem[pl.ds(prow, CH), :]
        o_store(i).start()

    for i in range(nch - NBUF, nch):
        o_store(i).wait()


def kernel(inputs, pos_table):
    B, S, D = inputs.shape
    x = inputs.reshape(B * S, D)
    out = pl.pallas_call(
        _body,
        in_specs=[
            pl.BlockSpec(memory_space=pl.ANY),
            pl.BlockSpec(memory_space=pl.ANY),
        ],
        out_specs=pl.BlockSpec(memory_space=pl.ANY),
        out_shape=jax.ShapeDtypeStruct((B * S, D), inputs.dtype),
        scratch_shapes=[
            pltpu.VMEM((S, D), jnp.float32),
            pltpu.VMEM((NBUF, CH, D), jnp.float32),
            pltpu.VMEM((NBUF, CH, D), jnp.float32),
            pltpu.SemaphoreType.DMA((NBUF,)),
            pltpu.SemaphoreType.DMA((NBUF,)),
            pltpu.SemaphoreType.DMA((S // CH,)),
        ],
    )(x, pos_table)
    return out.reshape(B, S, D)


# speedup vs baseline: 1.5110x; 1.0132x over previous
import jax
import jax.numpy as jnp
from jax.experimental import pallas as pl
from jax.experimental.pallas import tpu as pltpu

CH = 1024
NBUF = 5


def _body(x_hbm, p_hbm, o_hbm, pvmem, xbuf, obuf, xsem, osem, psem):
    N, D = x_hbm.shape
    S = p_hbm.shape[0]
    nch = N // CH
    pch = S // CH

    def p_load(j):
        return pltpu.make_async_copy(
            p_hbm.at[pl.ds(j * CH, CH)], pvmem.at[pl.ds(j * CH, CH)], psem.at[j]
        )

    def x_load(i):
        return pltpu.make_async_copy(
            x_hbm.at[pl.ds(i * CH, CH)], xbuf.at[i % NBUF], xsem.at[i % NBUF]
        )

    def o_store(i):
        return pltpu.make_async_copy(
            obuf.at[i % NBUF], o_hbm.at[pl.ds(i * CH, CH)], osem.at[i % NBUF]
        )

    for j in range(pch):
        p_load(j).start()
    for i in range(NBUF - 1):
        x_load(i).start()

    for i in range(nch):
        if i + NBUF - 1 < nch:
            x_load(i + NBUF - 1).start()
        x_load(i).wait()
        if i < pch:
            p_load(i).wait()
        if i >= NBUF:
            o_store(i - NBUF).wait()
        prow = (i * CH) % S
        obuf[i % NBUF] = xbuf[i % NBUF] + pvmem[pl.ds(prow, CH), :]
        o_store(i).start()

    for i in range(nch - NBUF, nch):
        o_store(i).wait()


def kernel(inputs, pos_table):
    B, S, D = inputs.shape
    x = inputs.reshape(B * S, D)
    out = pl.pallas_call(
        _body,
        in_specs=[
            pl.BlockSpec(memory_space=pl.ANY),
            pl.BlockSpec(memory_space=pl.ANY),
        ],
        out_specs=pl.BlockSpec(memory_space=pl.ANY),
        out_shape=jax.ShapeDtypeStruct((B * S, D), inputs.dtype),
        scratch_shapes=[
            pltpu.VMEM((S, D), jnp.float32),
            pltpu.VMEM((NBUF, CH, D), jnp.float32),
            pltpu.VMEM((NBUF, CH, D), jnp.float32),
            pltpu.SemaphoreType.DMA((NBUF,)),
            pltpu.SemaphoreType.DMA((NBUF,)),
            pltpu.SemaphoreType.DMA((S // CH,)),
        ],
    )(x, pos_table)
    return out.reshape(B, S, D)
